# Initial kernel scaffold; baseline (speedup 1.0000x reference)
import jax
import jax.numpy as jnp
from jax.experimental import pallas as pl


def kernel(tokens, emb, pos_emb):
    raise NotImplementedError("write your pallas kernel here")


# speedup vs baseline: 1.0455x; 1.0455x over previous
import functools

import jax
import jax.numpy as jnp
from jax import lax
from jax.experimental import pallas as pl
from jax.experimental.pallas import tpu as pltpu
from jax.experimental.pallas import tpu_sc as plsc

B = 4
S = 2048
D = 768
NC = 2
NS = 16
NW = NC * NS
P = S // NW
C = 32
H = P // C
LANES = 16
NCOL = D // LANES


def _body(tok_hbm, emb_hbm, pos_hbm, out_hbm, pos_v, idx_v, buf, sem):
    wid = lax.axis_index("s") * NC + lax.axis_index("c")
    p0 = wid * P
    pltpu.sync_copy(pos_hbm.at[pl.ds(p0, P)], pos_v)
    for b in range(B):
        for h in range(H):
            off = b * S + p0 + h * C
            pltpu.sync_copy(tok_hbm.at[pl.ds(off, C)], idx_v)
            pltpu.async_copy(emb_hbm.at[idx_v], buf, sem).wait()

            def addrow(r, carry, h=h):
                pr = h * C + r
                for c in range(NCOL):
                    sl = pl.ds(c * LANES, LANES)
                    buf[r, sl] = buf[r, sl] + pos_v[pr, sl]
                return carry

            lax.fori_loop(0, C, addrow, 0)
            pltpu.sync_copy(buf, out_hbm.at[b, pl.ds(p0 + h * C, C)])


@functools.partial(jax.jit, static_argnames=())
def _run(tok_flat, emb, pos_emb):
    mesh = plsc.VectorSubcoreMesh(core_axis_name="c", subcore_axis_name="s")
    f = pl.kernel(
        _body,
        out_type=jax.ShapeDtypeStruct((B, S, D), jnp.float32),
        mesh=mesh,
        scratch_types=[
            pltpu.VMEM((P, D), jnp.float32),
            pltpu.VMEM((C,), jnp.int32),
            pltpu.VMEM((C, D), jnp.float32),
            pltpu.SemaphoreType.DMA,
        ],
    )
    return f(tok_flat, emb, pos_emb)


def kernel(tokens, emb, pos_emb):
    tok_flat = tokens.reshape(-1).astype(jnp.int32)
    return _run(tok_flat, emb, pos_emb)
